# baseline (speedup 1.0000x reference)
import functools

import jax
import jax.numpy as jnp
from jax.experimental import pallas as pl

BLOCK_N = 2048


def _logits_kernel(f_ref, m_ref, o_ref):
    o_ref[...] = jax.lax.dot_general(
        f_ref[...],
        m_ref[...],
        dimension_numbers=(((1,), (1,)), ((), ())),
        preferred_element_type=jnp.float32,
    )


def kernel(feature, index, memory, temperature, momentum):
    b, k = feature.shape
    n = memory.shape[0]
    f_scaled = feature * (1.0 / temperature)
    grid = pl.cdiv(n, BLOCK_N)
    return pl.pallas_call(
        _logits_kernel,
        grid=(grid,),
        in_specs=[
            pl.BlockSpec((b, k), lambda i: (0, 0)),
            pl.BlockSpec((BLOCK_N, k), lambda i: (i, 0)),
        ],
        out_specs=pl.BlockSpec((b, BLOCK_N), lambda i: (0, i)),
        out_shape=jax.ShapeDtypeStruct((b, n), jnp.float32),
    )(f_scaled, memory)


# speedup vs baseline: 1.0019x; 1.0019x over previous
import functools

import jax
import jax.numpy as jnp
from jax.experimental import pallas as pl

BLOCK_N = 2048


def _logits_kernel(f_ref, m_ref, o_ref):
    o_ref[...] = jax.lax.dot_general(
        f_ref[...].astype(jnp.bfloat16),
        m_ref[...].astype(jnp.bfloat16),
        dimension_numbers=(((1,), (1,)), ((), ())),
        preferred_element_type=jnp.float32,
    )


def kernel(feature, index, memory, temperature, momentum):
    b, k = feature.shape
    n = memory.shape[0]
    f_scaled = feature * (1.0 / temperature)
    grid = pl.cdiv(n, BLOCK_N)
    return pl.pallas_call(
        _logits_kernel,
        grid=(grid,),
        in_specs=[
            pl.BlockSpec((b, k), lambda i: (0, 0)),
            pl.BlockSpec((BLOCK_N, k), lambda i: (i, 0)),
        ],
        out_specs=pl.BlockSpec((b, BLOCK_N), lambda i: (0, i)),
        out_shape=jax.ShapeDtypeStruct((b, n), jnp.float32),
    )(f_scaled, memory)
